# baseline (speedup 1.0000x reference)
import jax
import jax.numpy as jnp
from jax import lax
from jax.experimental import pallas as pl
from jax.experimental.pallas import tpu as pltpu
from jax.experimental.pallas import tpu_sc as plsc

N = 10000
E = 160000
D = 256
H = D // 2
CHUNK = 128
G8 = 8
NC = 2
NS = 16
NP = 10240
EP = 163840
NGROUPS = EP // (CHUNK * G8)
GROUPS_PER_TILE = NGROUPS // NS
CHUNKS_PER_TILE = GROUPS_PER_TILE * G8
ROWS_PER_TILE = NP // NS
NBUF = 2
LEAD = 1
NIB = 2
SUB = 32
NSUB = CHUNK // SUB


def _make_body():
    def body(xp_hbm, col2d_hbm, row2d_hbm, coln_hbm, out_hbm,
             coln_v, gidx_v, cidx0, cidx1, ridx0, ridx1,
             rows_v, acc_s,
             gsem, ssem, isem, csem):
        cidx = [cidx0, cidx1]
        ridx = [ridx0, ridx1]
        c_ax = lax.axis_index("c")
        s_ax = lax.axis_index("s")

        coln_cp = pltpu.async_copy(coln_hbm, coln_v, csem)

        def idx_dma(g):
            p = g % NIB
            jg = s_ax + g * NS
            a = pltpu.async_copy(col2d_hbm.at[pl.ds(jg * G8, G8)],
                                 cidx[p], isem[p])
            b = pltpu.async_copy(row2d_hbm.at[pl.ds(jg * G8, G8)],
                                 ridx[p], isem[p])
            return a, b

        idx_cp = {0: idx_dma(0)}

        def _zero_row(r, _):
            for k in range(H // 16):
                rows_v[r, 0, pl.ds(k * 16, 16)] = jnp.zeros((16,), jnp.float32)
            return 0
        lax.fori_loop(0, CHUNK, _zero_row, 0)
        for p in range(ROWS_PER_TILE // CHUNK):
            pltpu.sync_copy(
                rows_v.at[pl.ds(0, CHUNK)],
                acc_s.at[pl.ds(s_ax * ROWS_PER_TILE + p * CHUNK, CHUNK)])
        plsc.subcore_barrier()

        coln_cp.wait()
        off = jnp.full((16,), c_ax, jnp.int32)

        gather_cp = [None] * CHUNKS_PER_TILE
        scatter_cp = [None] * CHUNKS_PER_TILE

        def compute_gidx(g):
            p = g % NIB
            for r in range(G8):
                for k in range(CHUNK // 16):
                    ci = cidx[p][r, pl.ds(k * 16, 16)]
                    gv = plsc.load_gather(coln_v, [ci])
                    gidx_v[r, pl.ds(k * 16, 16)] = gv + gv + off

        def start_gather(t):
            b = t % NBUF
            if t >= NBUF:
                scatter_cp[t - NBUF].wait()
            c = t % G8
            gather_cp[t] = [
                pltpu.async_copy(
                    xp_hbm.at[gidx_v.at[c, pl.ds(q * SUB, SUB)]],
                    rows_v.at[pl.ds(b * CHUNK + q * SUB, SUB)],
                    gsem[b])
                for q in range(NSUB)]

        def start_scatter(t):
            b = t % NBUF
            g = t // G8
            c = t % G8
            for cp in gather_cp[t]:
                cp.wait()
            scatter_cp[t] = pltpu.async_copy(
                rows_v.at[pl.ds(b * CHUNK, CHUNK)],
                acc_s.at[ridx[g % NIB].at[c]], ssem[b], add=True)

        for g in range(GROUPS_PER_TILE):
            base = g * G8
            a, b = idx_cp[g]
            a.wait()
            b.wait()
            compute_gidx(g)
            for c in range(LEAD):
                start_gather(base + c)
            for c in range(G8):
                if c + LEAD < G8:
                    start_scatter(base + c)
                    start_gather(base + c + LEAD)
                else:
                    start_scatter(base + c)
                if c == LEAD and g + 1 < GROUPS_PER_TILE:
                    idx_cp[g + 1] = idx_dma(g + 1)

        for t in range(CHUNKS_PER_TILE - NBUF, CHUNKS_PER_TILE):
            scatter_cp[t].wait()
        plsc.subcore_barrier()

        last = N - (NS - 1) * ROWS_PER_TILE

        @pl.when(s_ax < NS - 1)
        def _():
            pltpu.sync_copy(
                acc_s.at[pl.ds(s_ax * ROWS_PER_TILE, ROWS_PER_TILE)],
                out_hbm.at[pl.ds(s_ax * ROWS_PER_TILE, ROWS_PER_TILE),
                           pl.ds(c_ax, 1)])

        @pl.when(s_ax == NS - 1)
        def _():
            pltpu.sync_copy(
                acc_s.at[pl.ds((NS - 1) * ROWS_PER_TILE, last)],
                out_hbm.at[pl.ds((NS - 1) * ROWS_PER_TILE, last),
                           pl.ds(c_ax, 1)])

    return body


@jax.jit
def kernel(x, edge_index):
    row = edge_index[0]
    col = edge_index[1]
    pe = EP - E
    col_p = jnp.concatenate([col, jnp.zeros((pe,), jnp.int32)])
    row_p = jnp.concatenate([row, jnp.full((pe,), N, jnp.int32)])
    col2d = col_p.reshape(EP // CHUNK, CHUNK)
    row2d = row_p.reshape(EP // CHUNK, CHUNK)
    coln = col[:N]
    xp = x.reshape(2 * N, 1, H)

    mesh = plsc.VectorSubcoreMesh(core_axis_name="c", subcore_axis_name="s",
                                  num_cores=NC, num_subcores=NS)
    out2 = pl.kernel(
        _make_body(),
        out_type=jax.ShapeDtypeStruct((N, NC, H), jnp.float32),
        mesh=mesh,
        compiler_params=pltpu.CompilerParams(needs_layout_passes=False),
        scratch_types=[
            pltpu.VMEM((N,), jnp.int32),
            pltpu.VMEM((G8, CHUNK), jnp.int32),
            pltpu.VMEM((G8, CHUNK), jnp.int32),
            pltpu.VMEM((G8, CHUNK), jnp.int32),
            pltpu.VMEM((G8, CHUNK), jnp.int32),
            pltpu.VMEM((G8, CHUNK), jnp.int32),
            pltpu.VMEM((NBUF * CHUNK, 1, H), jnp.float32),
            pltpu.VMEM_SHARED((NP, 1, H), jnp.float32),
            [pltpu.SemaphoreType.DMA] * NBUF,
            [pltpu.SemaphoreType.DMA] * NBUF,
            [pltpu.SemaphoreType.DMA] * NIB,
            pltpu.SemaphoreType.DMA,
        ],
    )(xp, col2d, row2d, coln)

    return out2.reshape(N, D)


# speedup vs baseline: 1.0846x; 1.0846x over previous
import jax
import jax.numpy as jnp
from jax import lax
from jax.experimental import pallas as pl
from jax.experimental.pallas import tpu as pltpu
from jax.experimental.pallas import tpu_sc as plsc

N = 10000
E = 160000
D = 256
H = D // 2
CHUNK = 128
G8 = 8
NC = 2
NS = 16
NP = 10240
EP = 163840
NGROUPS = EP // (CHUNK * G8)
GROUPS_PER_TILE = NGROUPS // NS
CHUNKS_PER_TILE = GROUPS_PER_TILE * G8
ROWS_PER_TILE = NP // NS
NBUF = 2
LEAD = 1
NIB = 2
SUB = 32
NSUB = CHUNK // SUB


def _make_body():
    def body(xp_hbm, col2d_hbm, row2d_hbm, coln_hbm, out_hbm,
             coln_v, gidx_v, cidx0, cidx1, ridx0, ridx1,
             rows_v, acc_s,
             gsem, ssem, isem, csem):
        cidx = [cidx0, cidx1]
        ridx = [ridx0, ridx1]
        c_ax = lax.axis_index("c")
        s_ax = lax.axis_index("s")

        coln_cp = pltpu.async_copy(coln_hbm, coln_v, csem)

        def idx_dma(g):
            p = g % NIB
            jg = s_ax + g * NS
            a = pltpu.async_copy(col2d_hbm.at[pl.ds(jg * G8, G8)],
                                 cidx[p], isem[p])
            b = pltpu.async_copy(row2d_hbm.at[pl.ds(jg * G8, G8)],
                                 ridx[p], isem[p])
            return a, b

        idx_cp = {0: idx_dma(0)}

        def _zero_row(r, _):
            for k in range(H // 16):
                rows_v[r, pl.ds(k * 16, 16)] = jnp.zeros((16,), jnp.float32)
            return 0
        lax.fori_loop(0, CHUNK, _zero_row, 0)
        for p in range(ROWS_PER_TILE // CHUNK):
            pltpu.sync_copy(
                rows_v.at[pl.ds(0, CHUNK)],
                acc_s.at[pl.ds(s_ax * ROWS_PER_TILE + p * CHUNK, CHUNK)])
        plsc.subcore_barrier()

        coln_cp.wait()
        off = jnp.full((16,), c_ax, jnp.int32)

        gather_cp = [None] * CHUNKS_PER_TILE
        scatter_cp = [None] * CHUNKS_PER_TILE

        def compute_gidx(g):
            p = g % NIB
            for r in range(G8):
                for k in range(CHUNK // 16):
                    ci = cidx[p][r, pl.ds(k * 16, 16)]
                    gv = plsc.load_gather(coln_v, [ci])
                    gidx_v[r, pl.ds(k * 16, 16)] = gv + gv + off

        def start_gather(t):
            b = t % NBUF
            if t >= NBUF:
                scatter_cp[t - NBUF].wait()
            c = t % G8
            gather_cp[t] = [
                pltpu.async_copy(
                    xp_hbm.at[gidx_v.at[c, pl.ds(q * SUB, SUB)]],
                    rows_v.at[pl.ds(b * CHUNK + q * SUB, SUB)],
                    gsem[b])
                for q in range(NSUB)]

        def start_scatter(t):
            b = t % NBUF
            g = t // G8
            c = t % G8
            for cp in gather_cp[t]:
                cp.wait()
            scatter_cp[t] = pltpu.async_copy(
                rows_v.at[pl.ds(b * CHUNK, CHUNK)],
                acc_s.at[ridx[g % NIB].at[c]], ssem[b], add=True)

        for g in range(GROUPS_PER_TILE):
            base = g * G8
            a, b = idx_cp[g]
            a.wait()
            b.wait()
            compute_gidx(g)
            for c in range(LEAD):
                start_gather(base + c)
            for c in range(G8):
                if c + LEAD < G8:
                    start_scatter(base + c)
                    start_gather(base + c + LEAD)
                else:
                    start_scatter(base + c)
                if c == LEAD and g + 1 < GROUPS_PER_TILE:
                    idx_cp[g + 1] = idx_dma(g + 1)

        for t in range(CHUNKS_PER_TILE - NBUF, CHUNKS_PER_TILE):
            scatter_cp[t].wait()
        plsc.subcore_barrier()

        pltpu.sync_copy(
            acc_s.at[pl.ds(s_ax * ROWS_PER_TILE, ROWS_PER_TILE)],
            out_hbm.at[pl.ds(s_ax * ROWS_PER_TILE, ROWS_PER_TILE),
                       pl.ds(c_ax * H, H)])

    return body


@jax.jit
def kernel(x, edge_index):
    row = edge_index[0]
    col = edge_index[1]
    pe = EP - E
    col_p = jnp.concatenate([col, jnp.zeros((pe,), jnp.int32)])
    row_p = jnp.concatenate([row, jnp.full((pe,), N, jnp.int32)])
    col2d = col_p.reshape(EP // CHUNK, CHUNK)
    row2d = row_p.reshape(EP // CHUNK, CHUNK)
    coln = col[:N]
    xp = x.reshape(2 * N, H)

    mesh = plsc.VectorSubcoreMesh(core_axis_name="c", subcore_axis_name="s",
                                  num_cores=NC, num_subcores=NS)
    out2 = pl.kernel(
        _make_body(),
        out_type=jax.ShapeDtypeStruct((NP, D), jnp.float32),
        mesh=mesh,
        compiler_params=pltpu.CompilerParams(needs_layout_passes=False),
        scratch_types=[
            pltpu.VMEM((N,), jnp.int32),
            pltpu.VMEM((G8, CHUNK), jnp.int32),
            pltpu.VMEM((G8, CHUNK), jnp.int32),
            pltpu.VMEM((G8, CHUNK), jnp.int32),
            pltpu.VMEM((G8, CHUNK), jnp.int32),
            pltpu.VMEM((G8, CHUNK), jnp.int32),
            pltpu.VMEM((NBUF * CHUNK, H), jnp.float32),
            pltpu.VMEM_SHARED((NP, H), jnp.float32),
            [pltpu.SemaphoreType.DMA] * NBUF,
            [pltpu.SemaphoreType.DMA] * NBUF,
            [pltpu.SemaphoreType.DMA] * NIB,
            pltpu.SemaphoreType.DMA,
        ],
    )(xp, col2d, row2d, coln)

    return out2[:N]


# speedup vs baseline: 1.1117x; 1.0250x over previous
import jax
import jax.numpy as jnp
from jax import lax
from jax.experimental import pallas as pl
from jax.experimental.pallas import tpu as pltpu
from jax.experimental.pallas import tpu_sc as plsc

N = 10000
E = 160000
D = 256
H = D // 2
CHUNK = 128
G8 = 8
NC = 2
NS = 16
NP = 10240
EP = 163840
NGROUPS = EP // (CHUNK * G8)
GROUPS_PER_TILE = NGROUPS // NS
CHUNKS_PER_TILE = GROUPS_PER_TILE * G8
ROWS_PER_TILE = NP // NS
NBUF = 2
LEAD = 1
NIB = 2
SUB = 32
NSUB = CHUNK // SUB


def _make_body():
    def body(xp_hbm, col2d_hbm, row2d_hbm, coln_hbm, out_hbm,
             coln_v, gidx_v, cidx0, cidx1, ridx0, ridx1,
             rows_v, acc_s,
             gsem, ssem, isem, csem):
        cidx = [cidx0, cidx1]
        ridx = [ridx0, ridx1]
        c_ax = lax.axis_index("c")
        s_ax = lax.axis_index("s")

        coln_cp = pltpu.async_copy(coln_hbm, coln_v, csem)

        def idx_dma(g):
            p = g % NIB
            jg = s_ax + g * NS
            a = pltpu.async_copy(col2d_hbm.at[pl.ds(jg * G8, G8)],
                                 cidx[p], isem[p])
            b = pltpu.async_copy(row2d_hbm.at[pl.ds(jg * G8, G8)],
                                 ridx[p], isem[p])
            return a, b

        idx_cp = {0: idx_dma(0)}

        def _zero_row(r, _):
            for k in range(H // 16):
                rows_v[r, pl.ds(k * 16, 16)] = jnp.zeros((16,), jnp.float32)
            return 0
        lax.fori_loop(0, CHUNK, _zero_row, 0)
        for p in range(ROWS_PER_TILE // CHUNK):
            pltpu.sync_copy(
                rows_v.at[pl.ds(0, CHUNK)],
                acc_s.at[pl.ds(s_ax * ROWS_PER_TILE + p * CHUNK, CHUNK)])
        plsc.subcore_barrier()

        coln_cp.wait()
        off = jnp.full((16,), c_ax, jnp.int32)

        gather_cp = [None] * CHUNKS_PER_TILE
        scatter_cp = [None] * CHUNKS_PER_TILE

        def compute_gidx(g):
            p = g % NIB
            for r in range(G8):
                for k in range(CHUNK // 16):
                    ci = cidx[p][r, pl.ds(k * 16, 16)]
                    gv = plsc.load_gather(coln_v, [ci])
                    gidx_v[r, pl.ds(k * 16, 16)] = gv + gv + off

        def start_gather(t):
            b = t % NBUF
            if t >= NBUF:
                scatter_cp[t - NBUF].wait()
            c = t % G8
            gather_cp[t] = [
                pltpu.async_copy(
                    xp_hbm.at[gidx_v.at[c, pl.ds(q * SUB, SUB)]],
                    rows_v.at[pl.ds(b * CHUNK + q * SUB, SUB)],
                    gsem[b])
                for q in range(NSUB)]

        def start_scatter(t):
            b = t % NBUF
            g = t // G8
            c = t % G8
            for cp in gather_cp[t]:
                cp.wait()
            scatter_cp[t] = pltpu.async_copy(
                rows_v.at[pl.ds(b * CHUNK, CHUNK)],
                acc_s.at[ridx[g % NIB].at[c]], ssem[b], add=True)

        for g in range(GROUPS_PER_TILE):
            base = g * G8
            a, b = idx_cp[g]
            a.wait()
            b.wait()
            compute_gidx(g)
            for c in range(LEAD):
                start_gather(base + c)
            for c in range(G8):
                if c + LEAD < G8:
                    start_scatter(base + c)
                    start_gather(base + c + LEAD)
                else:
                    start_scatter(base + c)
                if c == LEAD and g + 1 < GROUPS_PER_TILE:
                    idx_cp[g + 1] = idx_dma(g + 1)

        for t in range(CHUNKS_PER_TILE - NBUF, CHUNKS_PER_TILE):
            scatter_cp[t].wait()
        plsc.subcore_barrier()

        last = N - (NS - 1) * ROWS_PER_TILE

        @pl.when(s_ax < NS - 1)
        def _():
            pltpu.sync_copy(
                acc_s.at[pl.ds(s_ax * ROWS_PER_TILE, ROWS_PER_TILE)],
                out_hbm.at[pl.ds(s_ax * ROWS_PER_TILE, ROWS_PER_TILE),
                           pl.ds(c_ax * H, H)])

        @pl.when(s_ax == NS - 1)
        def _():
            pltpu.sync_copy(
                acc_s.at[pl.ds((NS - 1) * ROWS_PER_TILE, last)],
                out_hbm.at[pl.ds((NS - 1) * ROWS_PER_TILE, last),
                           pl.ds(c_ax * H, H)])

    return body


@jax.jit
def kernel(x, edge_index):
    row = edge_index[0]
    col = edge_index[1]
    pe = EP - E
    col_p = jnp.concatenate([col, jnp.zeros((pe,), jnp.int32)])
    row_p = jnp.concatenate([row, jnp.full((pe,), N, jnp.int32)])
    col2d = col_p.reshape(EP // CHUNK, CHUNK)
    row2d = row_p.reshape(EP // CHUNK, CHUNK)
    coln = col[:N]
    xp = x.reshape(2 * N, H)

    mesh = plsc.VectorSubcoreMesh(core_axis_name="c", subcore_axis_name="s",
                                  num_cores=NC, num_subcores=NS)
    out2 = pl.kernel(
        _make_body(),
        out_type=jax.ShapeDtypeStruct((N, D), jnp.float32),
        mesh=mesh,
        compiler_params=pltpu.CompilerParams(needs_layout_passes=False),
        scratch_types=[
            pltpu.VMEM((N,), jnp.int32),
            pltpu.VMEM((G8, CHUNK), jnp.int32),
            pltpu.VMEM((G8, CHUNK), jnp.int32),
            pltpu.VMEM((G8, CHUNK), jnp.int32),
            pltpu.VMEM((G8, CHUNK), jnp.int32),
            pltpu.VMEM((G8, CHUNK), jnp.int32),
            pltpu.VMEM((NBUF * CHUNK, H), jnp.float32),
            pltpu.VMEM_SHARED((NP, H), jnp.float32),
            [pltpu.SemaphoreType.DMA] * NBUF,
            [pltpu.SemaphoreType.DMA] * NBUF,
            [pltpu.SemaphoreType.DMA] * NIB,
            pltpu.SemaphoreType.DMA,
        ],
    )(xp, col2d, row2d, coln)

    return out2
